# baseline (speedup 1.0000x reference)
import functools

import jax
import jax.numpy as jnp
from jax.experimental import pallas as pl
from jax.experimental.pallas import tpu as pltpu

VOXEL_SIZE = 0.1
EPS = 1e-5
S = 512
R_BLK = 2048


def _gelu(x):
    return x * 0.5 * (1.0 + jax.lax.erf(x * 0.7071067811865476))


def _dot16(a, b):
    return jnp.dot(a.astype(jnp.bfloat16), b.astype(jnp.bfloat16),
                   preferred_element_type=jnp.float32)


def _bn_apply(x, stats, g, b, total):
    mu = stats[0:1, :] / total
    ex2 = stats[1:2, :] / total
    var = ex2 - mu * mu
    rstd = jax.lax.rsqrt(var + EPS)
    return (x - mu) * rstd * g + b


def _colstats(x):
    s = jnp.sum(x, axis=0, keepdims=True)
    s2 = jnp.sum(x * x, axis=0, keepdims=True)
    return jnp.concatenate([s, s2], axis=0)


def _k_prep(uh_ref, flat_ref, g4r_ref, w1_ref, idx_ref, norm_ref,
            stats_ref, acc):
    i = pl.program_id(0)
    uh = uh_ref[0, :]
    valid = uh >= 0
    vz = jnp.bitwise_and(uh, S - 1)
    q = jax.lax.shift_right_logical(uh, 9)
    vy = jnp.bitwise_and(q, S - 1)
    q = jax.lax.shift_right_logical(q, 9)
    vx = jnp.bitwise_and(q, S - 1)
    vb = jax.lax.shift_right_logical(q, 9)
    dec = jnp.stack([vb, vx, vy, vz], axis=0)
    idx_ref[...] = jnp.where(valid[None, :], dec, -1)

    g4 = g4r_ref[...]
    cent = g4[:, 0:3] / (g4[:, 3:4] + 1.0)
    norm = flat_ref[...] - cent
    norm_ref[...] = norm
    x1 = _dot16(norm, w1_ref[...])

    @pl.when(i == 0)
    def _():
        acc[...] = jnp.zeros_like(acc)

    acc[...] += _colstats(x1)
    stats_ref[...] = acc[...]


def _k_l12(norm_ref, w1_ref, stats1_ref, g1_ref, b1_ref, w2_ref,
           x2_ref, stats_ref, acc, *, total):
    i = pl.program_id(0)
    x1 = _dot16(norm_ref[...], w1_ref[...])
    p1 = _gelu(_bn_apply(x1, stats1_ref[...], g1_ref[...], b1_ref[...], total))
    x2 = _dot16(p1, w2_ref[...])
    x2_ref[...] = x2

    @pl.when(i == 0)
    def _():
        acc[...] = jnp.zeros_like(acc)

    acc[...] += _colstats(x2)
    stats_ref[...] = acc[...]


def _k_l3(x2_ref, stats2_ref, g2_ref, b2_ref, flat_ref, c4_ref, w3_ref,
          x3_ref, stats_ref, acc, *, total):
    i = pl.program_id(0)
    p2 = _gelu(_bn_apply(x2_ref[...], stats2_ref[...], g2_ref[...],
                         b2_ref[...], total))
    c4 = c4_ref[...]
    pooled = c4[:, 0:3] / (c4[:, 3:4] + 1.0)
    w3 = w3_ref[...]
    x3 = _dot16(p2, w3[3:259, :])
    x3 += _dot16(flat_ref[...], w3[0:3, :])
    x3 += _dot16(pooled, w3[259:262, :])
    x3_ref[...] = x3

    @pl.when(i == 0)
    def _():
        acc[...] = jnp.zeros_like(acc)

    acc[...] += _colstats(x3)
    stats_ref[...] = acc[...]


def _k_l4(x3_ref, stats3_ref, g3_ref, b3_ref, w4_ref, x4_ref, stats_ref,
          acc, *, total):
    i = pl.program_id(0)
    p3 = _gelu(_bn_apply(x3_ref[...], stats3_ref[...], g3_ref[...],
                         b3_ref[...], total))
    x4 = _dot16(p3, w4_ref[...])
    x4_ref[...] = x4

    @pl.when(i == 0)
    def _():
        acc[...] = jnp.zeros_like(acc)

    acc[...] += _colstats(x4)
    stats_ref[...] = acc[...]


def _k_final(x4_ref, stats4_ref, g4_ref, b4_ref, out_ref, *, total):
    out_ref[...] = _gelu(_bn_apply(x4_ref[...], stats4_ref[...],
                                   g4_ref[...], b4_ref[...], total))


def _row_spec(c):
    return pl.BlockSpec((R_BLK, c), lambda i: (i, 0))


def _full_spec(r, c):
    return pl.BlockSpec((r, c), lambda i: (0, 0))


def _mlp(uh1, flat, g4r, csum4, W1, g1, b1, W2, g2, b2, W3, g3, b3, W4,
         g4, b4, total):
    nb = total // R_BLK
    D = W2.shape[0]
    f32 = jnp.float32
    stats_sd = jax.ShapeDtypeStruct((2, D), f32)
    x_sd = jax.ShapeDtypeStruct((total, D), f32)
    scr = [pltpu.VMEM((2, D), f32)]
    g1r, b1r = g1[None, :], b1[None, :]
    g2r, b2r = g2[None, :], b2[None, :]
    g3r, b3r = g3[None, :], b3[None, :]
    g4rr, b4rr = g4[None, :], b4[None, :]

    idx, norm, stats1 = pl.pallas_call(
        _k_prep,
        grid=(nb,),
        in_specs=[pl.BlockSpec((1, R_BLK), lambda i: (0, i)), _row_spec(3),
                  _row_spec(4), _full_spec(3, D)],
        out_specs=[pl.BlockSpec((4, R_BLK), lambda i: (0, i)), _row_spec(3),
                   _full_spec(2, D)],
        out_shape=[jax.ShapeDtypeStruct((4, total), jnp.int32),
                   jax.ShapeDtypeStruct((total, 3), f32), stats_sd],
        scratch_shapes=scr,
    )(uh1[None, :], flat, g4r, W1)

    x2, stats2 = pl.pallas_call(
        functools.partial(_k_l12, total=float(total)),
        grid=(nb,),
        in_specs=[_row_spec(3), _full_spec(3, D), _full_spec(2, D),
                  _full_spec(1, D), _full_spec(1, D), _full_spec(D, D)],
        out_specs=[_row_spec(D), _full_spec(2, D)],
        out_shape=[x_sd, stats_sd],
        scratch_shapes=scr,
    )(norm, W1, stats1, g1r, b1r, W2)

    x3, stats3 = pl.pallas_call(
        functools.partial(_k_l3, total=float(total)),
        grid=(nb,),
        in_specs=[_row_spec(D), _full_spec(2, D), _full_spec(1, D),
                  _full_spec(1, D), _row_spec(3), _row_spec(4),
                  _full_spec(D + 6, D)],
        out_specs=[_row_spec(D), _full_spec(2, D)],
        out_shape=[x_sd, stats_sd],
        scratch_shapes=scr,
    )(x2, stats2, g2r, b2r, flat, csum4, W3)

    x4, stats4 = pl.pallas_call(
        functools.partial(_k_l4, total=float(total)),
        grid=(nb,),
        in_specs=[_row_spec(D), _full_spec(2, D), _full_spec(1, D),
                  _full_spec(1, D), _full_spec(D, D)],
        out_specs=[_row_spec(D), _full_spec(2, D)],
        out_shape=[x_sd, stats_sd],
        scratch_shapes=scr,
    )(x3, stats3, g3r, b3r, W4)

    agg = pl.pallas_call(
        functools.partial(_k_final, total=float(total)),
        grid=(nb,),
        in_specs=[_row_spec(D), _full_spec(2, D), _full_spec(1, D),
                  _full_spec(1, D)],
        out_specs=_row_spec(D),
        out_shape=x_sd,
    )(x4, stats4, g4rr, b4rr)

    return idx, norm, agg


def kernel(points, W1, g1, b1, W2, g2, b2, W3, g3, b3, W4, g4, b4):
    B, N, _ = points.shape
    total = B * N
    flat = points.reshape(-1, 3)
    pmin = jnp.min(flat, axis=0)
    vc = jnp.floor((flat - pmin) / VOXEL_SIZE).astype(jnp.int32)
    batch_ids = jnp.repeat(jnp.arange(B, dtype=jnp.int32), N)
    h1 = ((batch_ids * S + vc[:, 0]) * S + vc[:, 1]) * S + vc[:, 2]

    p1 = jnp.argsort(h1)
    s1 = h1[p1]
    f1 = jnp.concatenate([jnp.ones((1,), jnp.int32),
                          (s1[1:] != s1[:-1]).astype(jnp.int32)])
    r1 = jnp.cumsum(f1) - 1
    uh1 = jnp.full((total,), -1, jnp.int32).at[r1].set(s1)

    vc2 = jnp.floor(flat / VOXEL_SIZE).astype(jnp.int32)
    h2 = vc2[:, 0] * 73856093 + vc2[:, 1] * 19349663 + vc2[:, 2] * 83492791
    p2 = jnp.argsort(h2)
    s2 = h2[p2]
    f2 = jnp.concatenate([jnp.ones((1,), jnp.int32),
                          (s2[1:] != s2[:-1]).astype(jnp.int32)])
    r2 = jnp.cumsum(f2) - 1
    pts4 = jnp.concatenate([flat, jnp.ones((total, 1), jnp.float32)], axis=1)
    csum4 = jnp.zeros((total, 4), jnp.float32).at[r2].add(pts4[p2])
    g4r = csum4[p1]

    idx, norm, agg = _mlp(uh1, flat, g4r, csum4, W1, g1, b1, W2, g2, b2,
                          W3, g3, b3, W4, g4, b4, total)
    return idx, agg, norm


# speedup vs baseline: 1.2183x; 1.2183x over previous
import functools

import jax
import jax.numpy as jnp
from jax import lax
from jax.experimental import pallas as pl
from jax.experimental.pallas import tpu as pltpu
from jax.experimental.pallas import tpu_sc as plsc

VOXEL_SIZE = 0.1
EPS = 1e-5
S = 512
R_BLK = 2048
TOTAL = 65536
CHUNK = 4096

_sc_mesh = plsc.VectorSubcoreMesh(core_axis_name="c", subcore_axis_name="s")
_sc_params = pltpu.CompilerParams(needs_layout_passes=False)


@functools.partial(
    pl.kernel, mesh=_sc_mesh, compiler_params=_sc_params,
    out_type=[jax.ShapeDtypeStruct((TOTAL,), jnp.int32),
              jax.ShapeDtypeStruct((4 * TOTAL,), jnp.float32),
              jax.ShapeDtypeStruct((4 * TOTAL,), jnp.float32)],
    scratch_types=[pltpu.VMEM((32, 128), jnp.int32),
                   pltpu.VMEM((CHUNK,), jnp.int32),
                   pltpu.VMEM((CHUNK,), jnp.int32),
                   pltpu.VMEM((96, 128), jnp.int32),
                   pltpu.VMEM((96, 128), jnp.int32),
                   pltpu.VMEM((32, 128), jnp.int32),
                   pltpu.VMEM((12288,), jnp.float32),
                   pltpu.VMEM((CHUNK,), jnp.float32),
                   pltpu.VMEM((16384,), jnp.float32),
                   pltpu.VMEM((64, 128), jnp.int32),
                   pltpu.VMEM((8192,), jnp.float32),
                   pltpu.VMEM_SHARED((4 * TOTAL,), jnp.float32),
                   pltpu.SemaphoreType.DMA],
)
def _sc_segment(s1_hbm, r1_hbm, p1_hbm, p2_hbm, r2_hbm, pts_hbm,
                neg1_hbm, zeros_hbm, ones_hbm,
                uh1_hbm, csum_hbm, g4r_hbm,
                idxb, valb, pb, gidx, sidx, cidx, updb, onesb, rbb,
                g4i, g4b, sh, sem):
    cid = lax.axis_index("c")
    sid = lax.axis_index("s")
    base = sid * CHUNK

    @pl.when(cid == 0)
    def _():
        pltpu.sync_copy(neg1_hbm, valb)
        pltpu.sync_copy(valb, uh1_hbm.at[pl.ds(base, CHUNK)])

    @pl.when(cid == 1)
    def _():
        pltpu.sync_copy(zeros_hbm, sh.at[pl.ds(sid * 16384, 16384)])

    plsc.subcore_barrier()

    @pl.when(cid == 0)
    def _():
        pltpu.sync_copy(s1_hbm.at[pl.ds(base, CHUNK)], valb)
        pltpu.sync_copy(r1_hbm.at[pl.ds(sid * 32, 32)], idxb)

        def scat(j, _):
            pltpu.make_async_copy(valb.at[pl.ds(j * 128, 128)],
                                  uh1_hbm.at[idxb.at[j]], sem).start()
            return 0
        lax.fori_loop(0, 32, scat, 0)

        def drain(j, _):
            pltpu.make_async_copy(valb.at[pl.ds(j * 128, 128)],
                                  uh1_hbm.at[idxb.at[j]], sem).wait()
            return 0
        lax.fori_loop(0, 32, drain, 0)

    @pl.when(cid == 1)
    def _():
        pltpu.sync_copy(p2_hbm.at[pl.ds(base, CHUNK)], pb)

        def expand_g(m, _):
            q = m * 16 + lax.iota(jnp.int32, 16)
            i = q // 3
            g = plsc.load_gather(pb, [i])
            gidx[m // 8, pl.ds((m % 8) * 16, 16)] = g * 3 + (q - i * 3)
            return 0
        lax.fori_loop(0, 768, expand_g, 0)

        def gat(j, _):
            pltpu.make_async_copy(pts_hbm.at[gidx.at[j]],
                                  updb.at[pl.ds(j * 128, 128)], sem).start()
            return 0
        lax.fori_loop(0, 96, gat, 0)

        pltpu.sync_copy(r2_hbm.at[pl.ds(base, CHUNK)], pb)

        def expand_s(m, _):
            q = m * 16 + lax.iota(jnp.int32, 16)
            i = q // 3
            r = plsc.load_gather(pb, [i])
            sidx[m // 8, pl.ds((m % 8) * 16, 16)] = r * 4 + (q - i * 3)
            return 0
        lax.fori_loop(0, 768, expand_s, 0)

        def expand_c(m, _):
            r = pb[pl.ds(m * 16, 16)]
            cidx[m // 8, pl.ds((m % 8) * 16, 16)] = r * 4 + 3
            return 0
        lax.fori_loop(0, 256, expand_c, 0)
        pltpu.sync_copy(ones_hbm, onesb)

        def drain_g(j, _):
            pltpu.make_async_copy(pts_hbm.at[gidx.at[j]],
                                  updb.at[pl.ds(j * 128, 128)], sem).wait()
            return 0
        lax.fori_loop(0, 96, drain_g, 0)

        def add_xyz(j, _):
            pltpu.sync_copy(updb.at[pl.ds(j * 128, 128)],
                            sh.at[sidx.at[j]], add=True)
            return 0
        lax.fori_loop(0, 96, add_xyz, 0)

        def add_cnt(j, _):
            pltpu.sync_copy(onesb.at[pl.ds(j * 128, 128)],
                            sh.at[cidx.at[j]], add=True)
            return 0
        lax.fori_loop(0, 32, add_cnt, 0)

    plsc.subcore_barrier()

    @pl.when(cid == 1)
    def _():
        pltpu.sync_copy(sh.at[pl.ds(sid * 16384, 16384)], rbb)
        pltpu.sync_copy(rbb, csum_hbm.at[pl.ds(sid * 16384, 16384)])

    plsc.subcore_barrier()

    wid = sid * 2 + cid
    pltpu.sync_copy(p1_hbm.at[pl.ds(wid * 2048, 2048)], valb.at[pl.ds(0, 2048)])

    def expand_4(m, _):
        q = m * 16 + lax.iota(jnp.int32, 16)
        p = plsc.load_gather(valb, [jax.lax.shift_right_logical(q, 2)])
        g4i[m // 8, pl.ds((m % 8) * 16, 16)] = p * 4 + jnp.bitwise_and(q, 3)
        return 0
    lax.fori_loop(0, 512, expand_4, 0)

    def gat4(j, _):
        pltpu.make_async_copy(csum_hbm.at[g4i.at[j]],
                              g4b.at[pl.ds(j * 128, 128)], sem).start()
        return 0
    lax.fori_loop(0, 64, gat4, 0)

    def drain4(j, _):
        pltpu.make_async_copy(csum_hbm.at[g4i.at[j]],
                              g4b.at[pl.ds(j * 128, 128)], sem).wait()
        return 0
    lax.fori_loop(0, 64, drain4, 0)
    pltpu.sync_copy(g4b, g4r_hbm.at[pl.ds(wid * 8192, 8192)])


def _gelu(x):
    return x * 0.5 * (1.0 + jax.lax.erf(x * 0.7071067811865476))


def _dot16(a, b):
    return jnp.dot(a.astype(jnp.bfloat16), b.astype(jnp.bfloat16),
                   preferred_element_type=jnp.float32)


def _bn_apply(x, stats, g, b, total):
    mu = stats[0:1, :] / total
    ex2 = stats[1:2, :] / total
    var = ex2 - mu * mu
    rstd = jax.lax.rsqrt(var + EPS)
    return (x - mu) * rstd * g + b


def _colstats(x):
    s = jnp.sum(x, axis=0, keepdims=True)
    s2 = jnp.sum(x * x, axis=0, keepdims=True)
    return jnp.concatenate([s, s2], axis=0)


def _k_prep(uh_ref, flat_ref, g4r_ref, w1_ref, idx_ref, norm_ref,
            stats_ref, acc):
    i = pl.program_id(0)
    uh = uh_ref[0, :]
    valid = uh >= 0
    vz = jnp.bitwise_and(uh, S - 1)
    q = jax.lax.shift_right_logical(uh, 9)
    vy = jnp.bitwise_and(q, S - 1)
    q = jax.lax.shift_right_logical(q, 9)
    vx = jnp.bitwise_and(q, S - 1)
    vb = jax.lax.shift_right_logical(q, 9)
    dec = jnp.stack([vb, vx, vy, vz], axis=0)
    idx_ref[...] = jnp.where(valid[None, :], dec, -1)

    g4 = g4r_ref[...]
    cent = g4[:, 0:3] / (g4[:, 3:4] + 1.0)
    norm = flat_ref[...] - cent
    norm_ref[...] = norm
    x1 = _dot16(norm, w1_ref[...])

    @pl.when(i == 0)
    def _():
        acc[...] = jnp.zeros_like(acc)

    acc[...] += _colstats(x1)
    stats_ref[...] = acc[...]


def _k_l12(norm_ref, w1_ref, stats1_ref, g1_ref, b1_ref, w2_ref,
           x2_ref, stats_ref, acc, *, total):
    i = pl.program_id(0)
    x1 = _dot16(norm_ref[...], w1_ref[...])
    p1 = _gelu(_bn_apply(x1, stats1_ref[...], g1_ref[...], b1_ref[...], total))
    x2 = _dot16(p1, w2_ref[...])
    x2_ref[...] = x2

    @pl.when(i == 0)
    def _():
        acc[...] = jnp.zeros_like(acc)

    acc[...] += _colstats(x2)
    stats_ref[...] = acc[...]


def _k_l3(x2_ref, stats2_ref, g2_ref, b2_ref, flat_ref, c4_ref, w3_ref,
          x3_ref, stats_ref, acc, *, total):
    i = pl.program_id(0)
    p2 = _gelu(_bn_apply(x2_ref[...], stats2_ref[...], g2_ref[...],
                         b2_ref[...], total))
    c4 = c4_ref[...]
    pooled = c4[:, 0:3] / (c4[:, 3:4] + 1.0)
    w3 = w3_ref[...]
    x3 = _dot16(p2, w3[3:259, :])
    x3 += _dot16(flat_ref[...], w3[0:3, :])
    x3 += _dot16(pooled, w3[259:262, :])
    x3_ref[...] = x3

    @pl.when(i == 0)
    def _():
        acc[...] = jnp.zeros_like(acc)

    acc[...] += _colstats(x3)
    stats_ref[...] = acc[...]


def _k_l4(x3_ref, stats3_ref, g3_ref, b3_ref, w4_ref, x4_ref, stats_ref,
          acc, *, total):
    i = pl.program_id(0)
    p3 = _gelu(_bn_apply(x3_ref[...], stats3_ref[...], g3_ref[...],
                         b3_ref[...], total))
    x4 = _dot16(p3, w4_ref[...])
    x4_ref[...] = x4

    @pl.when(i == 0)
    def _():
        acc[...] = jnp.zeros_like(acc)

    acc[...] += _colstats(x4)
    stats_ref[...] = acc[...]


def _k_final(x4_ref, stats4_ref, g4_ref, b4_ref, out_ref, *, total):
    out_ref[...] = _gelu(_bn_apply(x4_ref[...], stats4_ref[...],
                                   g4_ref[...], b4_ref[...], total))


def _row_spec(c):
    return pl.BlockSpec((R_BLK, c), lambda i: (i, 0))


def _full_spec(r, c):
    return pl.BlockSpec((r, c), lambda i: (0, 0))


def _mlp(uh1, flat, g4r, csum4, W1, g1, b1, W2, g2, b2, W3, g3, b3, W4,
         g4, b4, total):
    nb = total // R_BLK
    D = W2.shape[0]
    f32 = jnp.float32
    stats_sd = jax.ShapeDtypeStruct((2, D), f32)
    x_sd = jax.ShapeDtypeStruct((total, D), f32)
    scr = [pltpu.VMEM((2, D), f32)]
    g1r, b1r = g1[None, :], b1[None, :]
    g2r, b2r = g2[None, :], b2[None, :]
    g3r, b3r = g3[None, :], b3[None, :]
    g4rr, b4rr = g4[None, :], b4[None, :]

    idx, norm, stats1 = pl.pallas_call(
        _k_prep,
        grid=(nb,),
        in_specs=[pl.BlockSpec((1, R_BLK), lambda i: (0, i)), _row_spec(3),
                  _row_spec(4), _full_spec(3, D)],
        out_specs=[pl.BlockSpec((4, R_BLK), lambda i: (0, i)), _row_spec(3),
                   _full_spec(2, D)],
        out_shape=[jax.ShapeDtypeStruct((4, total), jnp.int32),
                   jax.ShapeDtypeStruct((total, 3), f32), stats_sd],
        scratch_shapes=scr,
    )(uh1[None, :], flat, g4r, W1)

    x2, stats2 = pl.pallas_call(
        functools.partial(_k_l12, total=float(total)),
        grid=(nb,),
        in_specs=[_row_spec(3), _full_spec(3, D), _full_spec(2, D),
                  _full_spec(1, D), _full_spec(1, D), _full_spec(D, D)],
        out_specs=[_row_spec(D), _full_spec(2, D)],
        out_shape=[x_sd, stats_sd],
        scratch_shapes=scr,
    )(norm, W1, stats1, g1r, b1r, W2)

    x3, stats3 = pl.pallas_call(
        functools.partial(_k_l3, total=float(total)),
        grid=(nb,),
        in_specs=[_row_spec(D), _full_spec(2, D), _full_spec(1, D),
                  _full_spec(1, D), _row_spec(3), _row_spec(4),
                  _full_spec(D + 6, D)],
        out_specs=[_row_spec(D), _full_spec(2, D)],
        out_shape=[x_sd, stats_sd],
        scratch_shapes=scr,
    )(x2, stats2, g2r, b2r, flat, csum4, W3)

    x4, stats4 = pl.pallas_call(
        functools.partial(_k_l4, total=float(total)),
        grid=(nb,),
        in_specs=[_row_spec(D), _full_spec(2, D), _full_spec(1, D),
                  _full_spec(1, D), _full_spec(D, D)],
        out_specs=[_row_spec(D), _full_spec(2, D)],
        out_shape=[x_sd, stats_sd],
        scratch_shapes=scr,
    )(x3, stats3, g3r, b3r, W4)

    agg = pl.pallas_call(
        functools.partial(_k_final, total=float(total)),
        grid=(nb,),
        in_specs=[_row_spec(D), _full_spec(2, D), _full_spec(1, D),
                  _full_spec(1, D)],
        out_specs=_row_spec(D),
        out_shape=x_sd,
    )(x4, stats4, g4rr, b4rr)

    return idx, norm, agg


def kernel(points, W1, g1, b1, W2, g2, b2, W3, g3, b3, W4, g4, b4):
    B, N, _ = points.shape
    total = B * N
    flat = points.reshape(-1, 3)
    pmin = jnp.min(flat, axis=0)
    vc = jnp.floor((flat - pmin) / VOXEL_SIZE).astype(jnp.int32)
    batch_ids = jnp.repeat(jnp.arange(B, dtype=jnp.int32), N)
    h1 = ((batch_ids * S + vc[:, 0]) * S + vc[:, 1]) * S + vc[:, 2]

    iota = jnp.arange(total, dtype=jnp.int32)
    s1, p1 = lax.sort_key_val(h1, iota, is_stable=True)
    f1 = jnp.concatenate([jnp.ones((1,), jnp.int32),
                          (s1[1:] != s1[:-1]).astype(jnp.int32)])
    r1 = jnp.cumsum(f1) - 1

    vc2 = jnp.floor(flat / VOXEL_SIZE).astype(jnp.int32)
    h2 = vc2[:, 0] * 73856093 + vc2[:, 1] * 19349663 + vc2[:, 2] * 83492791
    s2, p2 = lax.sort_key_val(h2, iota, is_stable=True)
    f2 = jnp.concatenate([jnp.ones((1,), jnp.int32),
                          (s2[1:] != s2[:-1]).astype(jnp.int32)])
    r2 = jnp.cumsum(f2) - 1

    neg1 = jnp.full((CHUNK,), -1, jnp.int32)
    zeros = jnp.zeros((16384,), jnp.float32)
    ones = jnp.ones((CHUNK,), jnp.float32)
    uh1, csumf, g4rf = _sc_segment(s1, r1.reshape(512, 128), p1, p2, r2,
                                   flat.reshape(-1), neg1, zeros, ones)
    csum4 = csumf.reshape(total, 4)
    g4r = g4rf.reshape(total, 4)

    idx, norm, agg = _mlp(uh1, flat, g4r, csum4, W1, g1, b1, W2, g2, b2,
                          W3, g3, b3, W4, g4, b4, total)
    return idx, agg, norm


# speedup vs baseline: 1.2203x; 1.0017x over previous
import functools

import jax
import jax.numpy as jnp
from jax import lax
from jax.experimental import pallas as pl
from jax.experimental.pallas import tpu as pltpu
from jax.experimental.pallas import tpu_sc as plsc

VOXEL_SIZE = 0.1
EPS = 1e-5
S = 512
R_BLK = 2048
TOTAL = 65536
CHUNK = 4096

_sc_mesh = plsc.VectorSubcoreMesh(core_axis_name="c", subcore_axis_name="s")
_sc_params = pltpu.CompilerParams(needs_layout_passes=False)


@functools.partial(
    pl.kernel, mesh=_sc_mesh, compiler_params=_sc_params,
    out_type=[jax.ShapeDtypeStruct((TOTAL,), jnp.int32),
              jax.ShapeDtypeStruct((4 * TOTAL,), jnp.float32),
              jax.ShapeDtypeStruct((4 * TOTAL,), jnp.float32)],
    scratch_types=[pltpu.VMEM((32, 128), jnp.int32),
                   pltpu.VMEM((CHUNK,), jnp.int32),
                   pltpu.VMEM((CHUNK,), jnp.int32),
                   pltpu.VMEM((96, 128), jnp.int32),
                   pltpu.VMEM((96, 128), jnp.int32),
                   pltpu.VMEM((32, 128), jnp.int32),
                   pltpu.VMEM((12288,), jnp.float32),
                   pltpu.VMEM((CHUNK,), jnp.float32),
                   pltpu.VMEM((16384,), jnp.float32),
                   pltpu.VMEM((64, 128), jnp.int32),
                   pltpu.VMEM((8192,), jnp.float32),
                   pltpu.VMEM_SHARED((4 * TOTAL,), jnp.float32),
                   pltpu.SemaphoreType.DMA],
)
def _sc_segment(s1_hbm, r1_hbm, p1_hbm, p2_hbm, r2_hbm, pts_hbm,
                neg1_hbm, zeros_hbm, ones_hbm,
                uh1_hbm, csum_hbm, g4r_hbm,
                idxb, valb, pb, gidx, sidx, cidx, updb, onesb, rbb,
                g4i, g4b, sh, sem):
    cid = lax.axis_index("c")
    sid = lax.axis_index("s")
    base = sid * CHUNK

    @pl.when(cid == 0)
    def _():
        pltpu.sync_copy(neg1_hbm, valb)
        pltpu.sync_copy(valb, uh1_hbm.at[pl.ds(base, CHUNK)])

    @pl.when(cid == 1)
    def _():
        pltpu.sync_copy(zeros_hbm, sh.at[pl.ds(sid * 16384, 16384)])

    plsc.subcore_barrier()

    @pl.when(cid == 0)
    def _():
        pltpu.sync_copy(s1_hbm.at[pl.ds(base, CHUNK)], valb)
        pltpu.sync_copy(r1_hbm.at[pl.ds(sid * 32, 32)], idxb)

        def scat(j, _):
            pltpu.make_async_copy(valb.at[pl.ds(j * 128, 128)],
                                  uh1_hbm.at[idxb.at[j]], sem).start()
            return 0
        lax.fori_loop(0, 32, scat, 0)

        def drain(j, _):
            pltpu.make_async_copy(valb.at[pl.ds(j * 128, 128)],
                                  uh1_hbm.at[idxb.at[j]], sem).wait()
            return 0
        lax.fori_loop(0, 32, drain, 0)

    @pl.when(cid == 1)
    def _():
        pltpu.sync_copy(p2_hbm.at[pl.ds(base, CHUNK)], pb)

        def expand_g(m, _):
            q = m * 16 + lax.iota(jnp.int32, 16)
            i = q // 3
            g = plsc.load_gather(pb, [i])
            gidx[m // 8, pl.ds((m % 8) * 16, 16)] = g * 3 + (q - i * 3)
            return 0
        lax.fori_loop(0, 768, expand_g, 0)

        def gat(j, _):
            pltpu.make_async_copy(pts_hbm.at[gidx.at[j]],
                                  updb.at[pl.ds(j * 128, 128)], sem).start()
            return 0
        lax.fori_loop(0, 96, gat, 0)

        pltpu.sync_copy(r2_hbm.at[pl.ds(base, CHUNK)], pb)

        def expand_s(m, _):
            q = m * 16 + lax.iota(jnp.int32, 16)
            i = q // 3
            r = plsc.load_gather(pb, [i])
            sidx[m // 8, pl.ds((m % 8) * 16, 16)] = r * 4 + (q - i * 3)
            return 0
        lax.fori_loop(0, 768, expand_s, 0)

        def expand_c(m, _):
            r = pb[pl.ds(m * 16, 16)]
            cidx[m // 8, pl.ds((m % 8) * 16, 16)] = r * 4 + 3
            return 0
        lax.fori_loop(0, 256, expand_c, 0)
        pltpu.sync_copy(ones_hbm, onesb)

        def drain_g(j, _):
            pltpu.make_async_copy(pts_hbm.at[gidx.at[j]],
                                  updb.at[pl.ds(j * 128, 128)], sem).wait()
            return 0
        lax.fori_loop(0, 96, drain_g, 0)

        def add_xyz(j, _):
            pltpu.async_copy(updb.at[pl.ds(j * 128, 128)],
                             sh.at[sidx.at[j]], sem, add=True)
            return 0
        lax.fori_loop(0, 96, add_xyz, 0)

        def add_cnt(j, _):
            pltpu.async_copy(onesb.at[pl.ds(j * 128, 128)],
                             sh.at[cidx.at[j]], sem, add=True)
            return 0
        lax.fori_loop(0, 32, add_cnt, 0)

        def add_xyz_d(j, _):
            pltpu.make_async_copy(updb.at[pl.ds(j * 128, 128)],
                                  sh.at[sidx.at[j]], sem).wait()
            return 0
        lax.fori_loop(0, 96, add_xyz_d, 0)

        def add_cnt_d(j, _):
            pltpu.make_async_copy(onesb.at[pl.ds(j * 128, 128)],
                                  sh.at[cidx.at[j]], sem).wait()
            return 0
        lax.fori_loop(0, 32, add_cnt_d, 0)

    plsc.subcore_barrier()

    @pl.when(cid == 1)
    def _():
        pltpu.sync_copy(sh.at[pl.ds(sid * 16384, 16384)], rbb)
        pltpu.sync_copy(rbb, csum_hbm.at[pl.ds(sid * 16384, 16384)])

    plsc.subcore_barrier()

    wid = sid * 2 + cid
    pltpu.sync_copy(p1_hbm.at[pl.ds(wid * 2048, 2048)], valb.at[pl.ds(0, 2048)])

    def expand_4(m, _):
        q = m * 16 + lax.iota(jnp.int32, 16)
        p = plsc.load_gather(valb, [jax.lax.shift_right_logical(q, 2)])
        g4i[m // 8, pl.ds((m % 8) * 16, 16)] = p * 4 + jnp.bitwise_and(q, 3)
        return 0
    lax.fori_loop(0, 512, expand_4, 0)

    def gat4(j, _):
        pltpu.make_async_copy(csum_hbm.at[g4i.at[j]],
                              g4b.at[pl.ds(j * 128, 128)], sem).start()
        return 0
    lax.fori_loop(0, 64, gat4, 0)

    def drain4(j, _):
        pltpu.make_async_copy(csum_hbm.at[g4i.at[j]],
                              g4b.at[pl.ds(j * 128, 128)], sem).wait()
        return 0
    lax.fori_loop(0, 64, drain4, 0)
    pltpu.sync_copy(g4b, g4r_hbm.at[pl.ds(wid * 8192, 8192)])


def _gelu(x):
    return x * 0.5 * (1.0 + jax.lax.erf(x * 0.7071067811865476))


def _dot16(a, b):
    return jnp.dot(a.astype(jnp.bfloat16), b.astype(jnp.bfloat16),
                   preferred_element_type=jnp.float32)


def _bn_apply(x, stats, g, b, total):
    mu = stats[0:1, :] / total
    ex2 = stats[1:2, :] / total
    var = ex2 - mu * mu
    rstd = jax.lax.rsqrt(var + EPS)
    return (x - mu) * rstd * g + b


def _colstats(x):
    s = jnp.sum(x, axis=0, keepdims=True)
    s2 = jnp.sum(x * x, axis=0, keepdims=True)
    return jnp.concatenate([s, s2], axis=0)


def _k_prep(uh_ref, flat_ref, g4r_ref, w1_ref, idx_ref, norm_ref,
            stats_ref, acc):
    i = pl.program_id(0)
    uh = uh_ref[0, :]
    valid = uh >= 0
    vz = jnp.bitwise_and(uh, S - 1)
    q = jax.lax.shift_right_logical(uh, 9)
    vy = jnp.bitwise_and(q, S - 1)
    q = jax.lax.shift_right_logical(q, 9)
    vx = jnp.bitwise_and(q, S - 1)
    vb = jax.lax.shift_right_logical(q, 9)
    dec = jnp.stack([vb, vx, vy, vz], axis=0)
    idx_ref[...] = jnp.where(valid[None, :], dec, -1)

    g4 = g4r_ref[...]
    cent = g4[:, 0:3] / (g4[:, 3:4] + 1.0)
    norm = flat_ref[...] - cent
    norm_ref[...] = norm
    x1 = _dot16(norm, w1_ref[...])

    @pl.when(i == 0)
    def _():
        acc[...] = jnp.zeros_like(acc)

    acc[...] += _colstats(x1)
    stats_ref[...] = acc[...]


def _k_l12(norm_ref, w1_ref, stats1_ref, g1_ref, b1_ref, w2_ref,
           x2_ref, stats_ref, acc, *, total):
    i = pl.program_id(0)
    x1 = _dot16(norm_ref[...], w1_ref[...])
    p1 = _gelu(_bn_apply(x1, stats1_ref[...], g1_ref[...], b1_ref[...], total))
    x2 = _dot16(p1, w2_ref[...])
    x2_ref[...] = x2

    @pl.when(i == 0)
    def _():
        acc[...] = jnp.zeros_like(acc)

    acc[...] += _colstats(x2)
    stats_ref[...] = acc[...]


def _k_l3(x2_ref, stats2_ref, g2_ref, b2_ref, flat_ref, c4_ref, w3_ref,
          x3_ref, stats_ref, acc, *, total):
    i = pl.program_id(0)
    p2 = _gelu(_bn_apply(x2_ref[...], stats2_ref[...], g2_ref[...],
                         b2_ref[...], total))
    c4 = c4_ref[...]
    pooled = c4[:, 0:3] / (c4[:, 3:4] + 1.0)
    w3 = w3_ref[...]
    x3 = _dot16(p2, w3[3:259, :])
    x3 += _dot16(flat_ref[...], w3[0:3, :])
    x3 += _dot16(pooled, w3[259:262, :])
    x3_ref[...] = x3

    @pl.when(i == 0)
    def _():
        acc[...] = jnp.zeros_like(acc)

    acc[...] += _colstats(x3)
    stats_ref[...] = acc[...]


def _k_l4(x3_ref, stats3_ref, g3_ref, b3_ref, w4_ref, x4_ref, stats_ref,
          acc, *, total):
    i = pl.program_id(0)
    p3 = _gelu(_bn_apply(x3_ref[...], stats3_ref[...], g3_ref[...],
                         b3_ref[...], total))
    x4 = _dot16(p3, w4_ref[...])
    x4_ref[...] = x4

    @pl.when(i == 0)
    def _():
        acc[...] = jnp.zeros_like(acc)

    acc[...] += _colstats(x4)
    stats_ref[...] = acc[...]


def _k_final(x4_ref, stats4_ref, g4_ref, b4_ref, out_ref, *, total):
    out_ref[...] = _gelu(_bn_apply(x4_ref[...], stats4_ref[...],
                                   g4_ref[...], b4_ref[...], total))


def _row_spec(c):
    return pl.BlockSpec((R_BLK, c), lambda i: (i, 0))


def _full_spec(r, c):
    return pl.BlockSpec((r, c), lambda i: (0, 0))


def _mlp(uh1, flat, g4r, csum4, W1, g1, b1, W2, g2, b2, W3, g3, b3, W4,
         g4, b4, total):
    nb = total // R_BLK
    D = W2.shape[0]
    f32 = jnp.float32
    stats_sd = jax.ShapeDtypeStruct((2, D), f32)
    x_sd = jax.ShapeDtypeStruct((total, D), f32)
    scr = [pltpu.VMEM((2, D), f32)]
    g1r, b1r = g1[None, :], b1[None, :]
    g2r, b2r = g2[None, :], b2[None, :]
    g3r, b3r = g3[None, :], b3[None, :]
    g4rr, b4rr = g4[None, :], b4[None, :]

    idx, norm, stats1 = pl.pallas_call(
        _k_prep,
        grid=(nb,),
        in_specs=[pl.BlockSpec((1, R_BLK), lambda i: (0, i)), _row_spec(3),
                  _row_spec(4), _full_spec(3, D)],
        out_specs=[pl.BlockSpec((4, R_BLK), lambda i: (0, i)), _row_spec(3),
                   _full_spec(2, D)],
        out_shape=[jax.ShapeDtypeStruct((4, total), jnp.int32),
                   jax.ShapeDtypeStruct((total, 3), f32), stats_sd],
        scratch_shapes=scr,
    )(uh1[None, :], flat, g4r, W1)

    x2, stats2 = pl.pallas_call(
        functools.partial(_k_l12, total=float(total)),
        grid=(nb,),
        in_specs=[_row_spec(3), _full_spec(3, D), _full_spec(2, D),
                  _full_spec(1, D), _full_spec(1, D), _full_spec(D, D)],
        out_specs=[_row_spec(D), _full_spec(2, D)],
        out_shape=[x_sd, stats_sd],
        scratch_shapes=scr,
    )(norm, W1, stats1, g1r, b1r, W2)

    x3, stats3 = pl.pallas_call(
        functools.partial(_k_l3, total=float(total)),
        grid=(nb,),
        in_specs=[_row_spec(D), _full_spec(2, D), _full_spec(1, D),
                  _full_spec(1, D), _row_spec(3), _row_spec(4),
                  _full_spec(D + 6, D)],
        out_specs=[_row_spec(D), _full_spec(2, D)],
        out_shape=[x_sd, stats_sd],
        scratch_shapes=scr,
    )(x2, stats2, g2r, b2r, flat, csum4, W3)

    x4, stats4 = pl.pallas_call(
        functools.partial(_k_l4, total=float(total)),
        grid=(nb,),
        in_specs=[_row_spec(D), _full_spec(2, D), _full_spec(1, D),
                  _full_spec(1, D), _full_spec(D, D)],
        out_specs=[_row_spec(D), _full_spec(2, D)],
        out_shape=[x_sd, stats_sd],
        scratch_shapes=scr,
    )(x3, stats3, g3r, b3r, W4)

    agg = pl.pallas_call(
        functools.partial(_k_final, total=float(total)),
        grid=(nb,),
        in_specs=[_row_spec(D), _full_spec(2, D), _full_spec(1, D),
                  _full_spec(1, D)],
        out_specs=_row_spec(D),
        out_shape=x_sd,
    )(x4, stats4, g4rr, b4rr)

    return idx, norm, agg


def kernel(points, W1, g1, b1, W2, g2, b2, W3, g3, b3, W4, g4, b4):
    B, N, _ = points.shape
    total = B * N
    flat = points.reshape(-1, 3)
    pmin = jnp.min(flat, axis=0)
    vc = jnp.floor((flat - pmin) / VOXEL_SIZE).astype(jnp.int32)
    batch_ids = jnp.repeat(jnp.arange(B, dtype=jnp.int32), N)
    h1 = ((batch_ids * S + vc[:, 0]) * S + vc[:, 1]) * S + vc[:, 2]

    iota = jnp.arange(total, dtype=jnp.int32)
    s1, p1 = lax.sort_key_val(h1, iota, is_stable=True)
    f1 = jnp.concatenate([jnp.ones((1,), jnp.int32),
                          (s1[1:] != s1[:-1]).astype(jnp.int32)])
    r1 = jnp.cumsum(f1) - 1

    vc2 = jnp.floor(flat / VOXEL_SIZE).astype(jnp.int32)
    h2 = vc2[:, 0] * 73856093 + vc2[:, 1] * 19349663 + vc2[:, 2] * 83492791
    s2, p2 = lax.sort_key_val(h2, iota, is_stable=True)
    f2 = jnp.concatenate([jnp.ones((1,), jnp.int32),
                          (s2[1:] != s2[:-1]).astype(jnp.int32)])
    r2 = jnp.cumsum(f2) - 1

    neg1 = jnp.full((CHUNK,), -1, jnp.int32)
    zeros = jnp.zeros((16384,), jnp.float32)
    ones = jnp.ones((CHUNK,), jnp.float32)
    uh1, csumf, g4rf = _sc_segment(s1, r1.reshape(512, 128), p1, p2, r2,
                                   flat.reshape(-1), neg1, zeros, ones)
    csum4 = csumf.reshape(total, 4)
    g4r = g4rf.reshape(total, 4)

    idx, norm, agg = _mlp(uh1, flat, g4r, csum4, W1, g1, b1, W2, g2, b2,
                          W3, g3, b3, W4, g4, b4, total)
    return idx, agg, norm


# speedup vs baseline: 1.2478x; 1.0225x over previous
import functools

import jax
import jax.numpy as jnp
from jax import lax
from jax.experimental import pallas as pl
from jax.experimental.pallas import tpu as pltpu
from jax.experimental.pallas import tpu_sc as plsc

VOXEL_SIZE = 0.1
EPS = 1e-5
S = 512
R_BLK = 2048
TOTAL = 65536
CHUNK = 4096

_sc_mesh = plsc.VectorSubcoreMesh(core_axis_name="c", subcore_axis_name="s")
_sc_params = pltpu.CompilerParams(needs_layout_passes=False)


@functools.partial(
    pl.kernel, mesh=_sc_mesh, compiler_params=_sc_params,
    out_type=[jax.ShapeDtypeStruct((TOTAL,), jnp.int32),
              jax.ShapeDtypeStruct((4 * TOTAL,), jnp.float32),
              jax.ShapeDtypeStruct((4 * TOTAL,), jnp.float32)],
    scratch_types=[pltpu.VMEM((32, 128), jnp.int32),
                   pltpu.VMEM((CHUNK,), jnp.int32),
                   pltpu.VMEM((CHUNK,), jnp.int32),
                   pltpu.VMEM((96, 128), jnp.int32),
                   pltpu.VMEM((96, 128), jnp.int32),
                   pltpu.VMEM((32, 128), jnp.int32),
                   pltpu.VMEM((12288,), jnp.float32),
                   pltpu.VMEM((CHUNK,), jnp.float32),
                   pltpu.VMEM((16384,), jnp.float32),
                   pltpu.VMEM((64, 128), jnp.int32),
                   pltpu.VMEM((8192,), jnp.float32),
                   pltpu.VMEM_SHARED((4 * TOTAL,), jnp.float32),
                   pltpu.SemaphoreType.DMA],
)
def _sc_segment(s1_hbm, r1_hbm, p1_hbm, p2_hbm, r2_hbm, pts_hbm,
                neg1_hbm, zeros_hbm, ones_hbm,
                uh1_hbm, csum_hbm, g4r_hbm,
                idxb, valb, pb, gidx, sidx, cidx, updb, onesb, rbb,
                g4i, g4b, sh, sem):
    cid = lax.axis_index("c")
    sid = lax.axis_index("s")
    base = sid * CHUNK

    @pl.when(cid == 0)
    def _():
        pltpu.sync_copy(neg1_hbm, valb)
        pltpu.sync_copy(valb, uh1_hbm.at[pl.ds(base, CHUNK)])

    @pl.when(cid == 1)
    def _():
        pltpu.sync_copy(zeros_hbm, sh.at[pl.ds(sid * 16384, 16384)])

    plsc.subcore_barrier()

    @pl.when(cid == 0)
    def _():
        pltpu.sync_copy(s1_hbm.at[pl.ds(base, CHUNK)], valb)
        pltpu.sync_copy(r1_hbm.at[pl.ds(sid * 32, 32)], idxb)

        def scat(j, _):
            pltpu.make_async_copy(valb.at[pl.ds(j * 128, 128)],
                                  uh1_hbm.at[idxb.at[j]], sem).start()
            return 0
        lax.fori_loop(0, 32, scat, 0)

        def drain(j, _):
            pltpu.make_async_copy(valb.at[pl.ds(j * 128, 128)],
                                  uh1_hbm.at[idxb.at[j]], sem).wait()
            return 0
        lax.fori_loop(0, 32, drain, 0)

    @pl.when(cid == 1)
    def _():
        pltpu.sync_copy(p2_hbm.at[pl.ds(base, CHUNK)], pb)

        def expand_g(m, _):
            q = m * 16 + lax.iota(jnp.int32, 16)
            i = q // 3
            g = plsc.load_gather(pb, [i])
            gidx[m // 8, pl.ds((m % 8) * 16, 16)] = g * 3 + (q - i * 3)
            return 0
        lax.fori_loop(0, 768, expand_g, 0)

        def gat(j, _):
            pltpu.make_async_copy(pts_hbm.at[gidx.at[j]],
                                  updb.at[pl.ds(j * 128, 128)], sem).start()
            return 0
        lax.fori_loop(0, 96, gat, 0)

        pltpu.sync_copy(r2_hbm.at[pl.ds(base, CHUNK)], pb)

        def expand_s(m, _):
            q = m * 16 + lax.iota(jnp.int32, 16)
            i = q // 3
            r = plsc.load_gather(pb, [i])
            sidx[m // 8, pl.ds((m % 8) * 16, 16)] = r * 4 + (q - i * 3)
            return 0
        lax.fori_loop(0, 768, expand_s, 0)

        def expand_c(m, _):
            r = pb[pl.ds(m * 16, 16)]
            cidx[m // 8, pl.ds((m % 8) * 16, 16)] = r * 4 + 3
            return 0
        lax.fori_loop(0, 256, expand_c, 0)
        pltpu.sync_copy(ones_hbm, onesb)

        def drain_g(j, _):
            pltpu.make_async_copy(pts_hbm.at[gidx.at[j]],
                                  updb.at[pl.ds(j * 128, 128)], sem).wait()
            return 0
        lax.fori_loop(0, 96, drain_g, 0)

        def add_xyz(j, _):
            pltpu.async_copy(updb.at[pl.ds(j * 128, 128)],
                             sh.at[sidx.at[j]], sem, add=True)
            return 0
        lax.fori_loop(0, 96, add_xyz, 0)

        def add_cnt(j, _):
            pltpu.async_copy(onesb.at[pl.ds(j * 128, 128)],
                             sh.at[cidx.at[j]], sem, add=True)
            return 0
        lax.fori_loop(0, 32, add_cnt, 0)

        def add_xyz_d(j, _):
            pltpu.make_async_copy(updb.at[pl.ds(j * 128, 128)],
                                  sh.at[sidx.at[j]], sem).wait()
            return 0
        lax.fori_loop(0, 96, add_xyz_d, 0)

        def add_cnt_d(j, _):
            pltpu.make_async_copy(onesb.at[pl.ds(j * 128, 128)],
                                  sh.at[cidx.at[j]], sem).wait()
            return 0
        lax.fori_loop(0, 32, add_cnt_d, 0)

    plsc.subcore_barrier()

    @pl.when(cid == 1)
    def _():
        pltpu.sync_copy(sh.at[pl.ds(sid * 16384, 16384)], rbb)
        pltpu.sync_copy(rbb, csum_hbm.at[pl.ds(sid * 16384, 16384)])

    plsc.subcore_barrier()

    wid = sid * 2 + cid
    pltpu.sync_copy(p1_hbm.at[pl.ds(wid * 2048, 2048)], valb.at[pl.ds(0, 2048)])

    def expand_4(m, _):
        q = m * 16 + lax.iota(jnp.int32, 16)
        p = plsc.load_gather(valb, [jax.lax.shift_right_logical(q, 2)])
        g4i[m // 8, pl.ds((m % 8) * 16, 16)] = p * 4 + jnp.bitwise_and(q, 3)
        return 0
    lax.fori_loop(0, 512, expand_4, 0)

    def gat4(j, _):
        pltpu.make_async_copy(csum_hbm.at[g4i.at[j]],
                              g4b.at[pl.ds(j * 128, 128)], sem).start()
        return 0
    lax.fori_loop(0, 64, gat4, 0)

    def drain4(j, _):
        pltpu.make_async_copy(csum_hbm.at[g4i.at[j]],
                              g4b.at[pl.ds(j * 128, 128)], sem).wait()
        return 0
    lax.fori_loop(0, 64, drain4, 0)
    pltpu.sync_copy(g4b, g4r_hbm.at[pl.ds(wid * 8192, 8192)])


def _gelu(x):
    return x * 0.5 * (1.0 + jax.lax.erf(x * 0.7071067811865476))


def _dot16(a, b):
    return jnp.dot(a.astype(jnp.bfloat16), b.astype(jnp.bfloat16),
                   preferred_element_type=jnp.float32)


def _bn_apply(x, stats, g, b, total):
    mu = stats[0:1, :] / total
    ex2 = stats[1:2, :] / total
    var = ex2 - mu * mu
    rstd = jax.lax.rsqrt(var + EPS)
    return (x - mu) * rstd * g + b


def _colstats(x):
    s = jnp.sum(x, axis=0, keepdims=True)
    s2 = jnp.sum(x * x, axis=0, keepdims=True)
    return jnp.concatenate([s, s2], axis=0)


def _k_prep(uh_ref, flat_ref, g4r_ref, w1_ref, idx_ref, norm_ref,
            stats_ref, acc):
    i = pl.program_id(0)
    uh = uh_ref[0, :]
    valid = uh >= 0
    vz = jnp.bitwise_and(uh, S - 1)
    q = jax.lax.shift_right_logical(uh, 9)
    vy = jnp.bitwise_and(q, S - 1)
    q = jax.lax.shift_right_logical(q, 9)
    vx = jnp.bitwise_and(q, S - 1)
    vb = jax.lax.shift_right_logical(q, 9)
    dec = jnp.stack([vb, vx, vy, vz], axis=0)
    idx_ref[...] = jnp.where(valid[None, :], dec, -1)

    g4 = g4r_ref[...]
    cent = g4[:, 0:3] / (g4[:, 3:4] + 1.0)
    norm = flat_ref[...] - cent
    norm_ref[...] = norm
    x1 = _dot16(norm, w1_ref[...])

    @pl.when(i == 0)
    def _():
        acc[...] = jnp.zeros_like(acc)

    acc[...] += _colstats(x1)
    stats_ref[...] = acc[...]


def _k_l12(norm_ref, w1_ref, stats1_ref, g1_ref, b1_ref, w2_ref,
           x2_ref, stats_ref, acc, *, total):
    i = pl.program_id(0)
    x1 = _dot16(norm_ref[...], w1_ref[...])
    p1 = _gelu(_bn_apply(x1, stats1_ref[...], g1_ref[...], b1_ref[...], total))
    x2 = _dot16(p1, w2_ref[...])
    x2_ref[...] = x2.astype(jnp.bfloat16)

    @pl.when(i == 0)
    def _():
        acc[...] = jnp.zeros_like(acc)

    acc[...] += _colstats(x2)
    stats_ref[...] = acc[...]


def _k_l3(x2_ref, stats2_ref, g2_ref, b2_ref, flat_ref, c4_ref, w3_ref,
          x3_ref, stats_ref, acc, *, total):
    i = pl.program_id(0)
    p2 = _gelu(_bn_apply(x2_ref[...], stats2_ref[...], g2_ref[...],
                         b2_ref[...], total))
    c4 = c4_ref[...]
    pooled = c4[:, 0:3] / (c4[:, 3:4] + 1.0)
    w3 = w3_ref[...]
    x3 = _dot16(p2, w3[3:259, :])
    x3 += _dot16(flat_ref[...], w3[0:3, :])
    x3 += _dot16(pooled, w3[259:262, :])
    x3_ref[...] = x3.astype(jnp.bfloat16)

    @pl.when(i == 0)
    def _():
        acc[...] = jnp.zeros_like(acc)

    acc[...] += _colstats(x3)
    stats_ref[...] = acc[...]


def _k_l4(x3_ref, stats3_ref, g3_ref, b3_ref, w4_ref, x4_ref, stats_ref,
          acc, *, total):
    i = pl.program_id(0)
    p3 = _gelu(_bn_apply(x3_ref[...], stats3_ref[...], g3_ref[...],
                         b3_ref[...], total))
    x4 = _dot16(p3, w4_ref[...])
    x4_ref[...] = x4.astype(jnp.bfloat16)

    @pl.when(i == 0)
    def _():
        acc[...] = jnp.zeros_like(acc)

    acc[...] += _colstats(x4)
    stats_ref[...] = acc[...]


def _k_final(x4_ref, stats4_ref, g4_ref, b4_ref, out_ref, *, total):
    out_ref[...] = _gelu(_bn_apply(x4_ref[...], stats4_ref[...],
                                   g4_ref[...], b4_ref[...], total))


def _row_spec(c):
    return pl.BlockSpec((R_BLK, c), lambda i: (i, 0))


def _full_spec(r, c):
    return pl.BlockSpec((r, c), lambda i: (0, 0))


def _mlp(uh1, flat, g4r, csum4, W1, g1, b1, W2, g2, b2, W3, g3, b3, W4,
         g4, b4, total):
    nb = total // R_BLK
    D = W2.shape[0]
    f32 = jnp.float32
    stats_sd = jax.ShapeDtypeStruct((2, D), f32)
    x_sd = jax.ShapeDtypeStruct((total, D), jnp.bfloat16)
    xf_sd = jax.ShapeDtypeStruct((total, D), f32)
    scr = [pltpu.VMEM((2, D), f32)]
    g1r, b1r = g1[None, :], b1[None, :]
    g2r, b2r = g2[None, :], b2[None, :]
    g3r, b3r = g3[None, :], b3[None, :]
    g4rr, b4rr = g4[None, :], b4[None, :]

    idx, norm, stats1 = pl.pallas_call(
        _k_prep,
        grid=(nb,),
        in_specs=[pl.BlockSpec((1, R_BLK), lambda i: (0, i)), _row_spec(3),
                  _row_spec(4), _full_spec(3, D)],
        out_specs=[pl.BlockSpec((4, R_BLK), lambda i: (0, i)), _row_spec(3),
                   _full_spec(2, D)],
        out_shape=[jax.ShapeDtypeStruct((4, total), jnp.int32),
                   jax.ShapeDtypeStruct((total, 3), f32), stats_sd],
        scratch_shapes=scr,
    )(uh1[None, :], flat, g4r, W1)

    x2, stats2 = pl.pallas_call(
        functools.partial(_k_l12, total=float(total)),
        grid=(nb,),
        in_specs=[_row_spec(3), _full_spec(3, D), _full_spec(2, D),
                  _full_spec(1, D), _full_spec(1, D), _full_spec(D, D)],
        out_specs=[_row_spec(D), _full_spec(2, D)],
        out_shape=[x_sd, stats_sd],
        scratch_shapes=scr,
    )(norm, W1, stats1, g1r, b1r, W2)

    x3, stats3 = pl.pallas_call(
        functools.partial(_k_l3, total=float(total)),
        grid=(nb,),
        in_specs=[_row_spec(D), _full_spec(2, D), _full_spec(1, D),
                  _full_spec(1, D), _row_spec(3), _row_spec(4),
                  _full_spec(D + 6, D)],
        out_specs=[_row_spec(D), _full_spec(2, D)],
        out_shape=[x_sd, stats_sd],
        scratch_shapes=scr,
    )(x2, stats2, g2r, b2r, flat, csum4, W3)

    x4, stats4 = pl.pallas_call(
        functools.partial(_k_l4, total=float(total)),
        grid=(nb,),
        in_specs=[_row_spec(D), _full_spec(2, D), _full_spec(1, D),
                  _full_spec(1, D), _full_spec(D, D)],
        out_specs=[_row_spec(D), _full_spec(2, D)],
        out_shape=[x_sd, stats_sd],
        scratch_shapes=scr,
    )(x3, stats3, g3r, b3r, W4)

    agg = pl.pallas_call(
        functools.partial(_k_final, total=float(total)),
        grid=(nb,),
        in_specs=[_row_spec(D), _full_spec(2, D), _full_spec(1, D),
                  _full_spec(1, D)],
        out_specs=_row_spec(D),
        out_shape=xf_sd,
    )(x4, stats4, g4rr, b4rr)

    return idx, norm, agg


def kernel(points, W1, g1, b1, W2, g2, b2, W3, g3, b3, W4, g4, b4):
    B, N, _ = points.shape
    total = B * N
    flat = points.reshape(-1, 3)
    pmin = jnp.min(flat, axis=0)
    vc = jnp.floor((flat - pmin) / VOXEL_SIZE).astype(jnp.int32)
    batch_ids = jnp.repeat(jnp.arange(B, dtype=jnp.int32), N)
    h1 = ((batch_ids * S + vc[:, 0]) * S + vc[:, 1]) * S + vc[:, 2]

    iota = jnp.arange(total, dtype=jnp.int32)
    s1, p1 = lax.sort_key_val(h1, iota, is_stable=True)
    f1 = jnp.concatenate([jnp.ones((1,), jnp.int32),
                          (s1[1:] != s1[:-1]).astype(jnp.int32)])
    r1 = jnp.cumsum(f1) - 1

    vc2 = jnp.floor(flat / VOXEL_SIZE).astype(jnp.int32)
    h2 = vc2[:, 0] * 73856093 + vc2[:, 1] * 19349663 + vc2[:, 2] * 83492791
    s2, p2 = lax.sort_key_val(h2, iota, is_stable=True)
    f2 = jnp.concatenate([jnp.ones((1,), jnp.int32),
                          (s2[1:] != s2[:-1]).astype(jnp.int32)])
    r2 = jnp.cumsum(f2) - 1

    neg1 = jnp.full((CHUNK,), -1, jnp.int32)
    zeros = jnp.zeros((16384,), jnp.float32)
    ones = jnp.ones((CHUNK,), jnp.float32)
    uh1, csumf, g4rf = _sc_segment(s1, r1.reshape(512, 128), p1, p2, r2,
                                   flat.reshape(-1), neg1, zeros, ones)
    csum4 = csumf.reshape(total, 4)
    g4r = g4rf.reshape(total, 4)

    idx, norm, agg = _mlp(uh1, flat, g4r, csum4, W1, g1, b1, W2, g2, b2,
                          W3, g3, b3, W4, g4, b4, total)
    return idx, agg, norm
